# Initial kernel scaffold; baseline (speedup 1.0000x reference)
import jax
import jax.numpy as jnp
from jax.experimental import pallas as pl


def kernel(orig_atom_fea, nbr_fea, self_fea_idx, nbr_fea_idx, crystal_atom_idx, emb_W, emb_b, filt_W, filt_b, core_W, core_b, bnf_g, bnf_b, bnc_g, bnc_b, bno_g, bno_b, gtf_W, gtf_b, gtfbn_g, gtfbn_b, out_W, out_b):
    raise NotImplementedError("write your pallas kernel here")


# speedup vs baseline: 1.7788x; 1.7788x over previous
import functools

import jax
import jax.numpy as jnp
from jax import lax
from jax.experimental import pallas as pl
from jax.experimental.pallas import tpu as pltpu
from jax.experimental.pallas import tpu_sc as plsc

N = 100000
E = 1600000
C = 256
ORIG = 128
ATOM = 48
NBR = 16
NG = 3
H = 128
W56 = 56
EPS = 1e-5

BN_NODE = 2000
BE = 2000


def _softplus(x):
    return jnp.log1p(jnp.exp(-jnp.abs(x))) + jnp.maximum(x, 0.0)


def _sigmoid(x):
    return 1.0 / (1.0 + jnp.exp(-x))


def _embed_body(x_ref, w_ref, b_ref, o_ref):
    o_ref[...] = jnp.dot(x_ref[...], w_ref[...],
                         preferred_element_type=jnp.float32) + b_ref[0:1, :]


def _embed(orig, embWT, b8):
    grid = N // BN_NODE
    return pl.pallas_call(
        _embed_body,
        grid=(grid,),
        in_specs=[
            pl.BlockSpec((BN_NODE, ORIG), lambda j: (j, 0)),
            pl.BlockSpec((ORIG, ATOM), lambda j: (0, 0)),
            pl.BlockSpec((8, ATOM), lambda j: (0, 0)),
        ],
        out_specs=pl.BlockSpec((BN_NODE, ATOM), lambda j: (j, 0)),
        out_shape=jax.ShapeDtypeStruct((N, ATOM), jnp.float32),
    )(orig, embWT, b8)


def _passA_body(gs_ref, gn_ref, nbr_ref, ws_ref, wn_ref, wf_ref,
                fc_ref, st_ref, acc_ref):
    pi = pl.program_id(0)
    r = (jnp.dot(gs_ref[...], ws_ref[...], preferred_element_type=jnp.float32)
         + jnp.dot(gn_ref[...], wn_ref[...], preferred_element_type=jnp.float32)
         + jnp.dot(nbr_ref[...], wf_ref[...], preferred_element_type=jnp.float32))
    fc_ref[...] = r

    @pl.when(pi == 0)
    def _():
        acc_ref[...] = jnp.zeros_like(acc_ref)

    acc_ref[0:1, :] += jnp.sum(r, axis=0, keepdims=True)
    acc_ref[1:2, :] += jnp.sum(r * r, axis=0, keepdims=True)

    @pl.when(pi == pl.num_programs(0) - 1)
    def _():
        st_ref[...] = acc_ref[...]


def _passA(gs, gn, nbr, wsT, wnT, wfT):
    grid = E // BE
    return pl.pallas_call(
        _passA_body,
        grid=(grid,),
        in_specs=[
            pl.BlockSpec((BE, ATOM), lambda j: (j, 0)),
            pl.BlockSpec((BE, ATOM), lambda j: (j, 0)),
            pl.BlockSpec((BE, NBR), lambda j: (j, 0)),
            pl.BlockSpec((ATOM, 96), lambda j: (0, 0)),
            pl.BlockSpec((ATOM, 96), lambda j: (0, 0)),
            pl.BlockSpec((NBR, 96), lambda j: (0, 0)),
        ],
        out_specs=[
            pl.BlockSpec((BE, 96), lambda j: (j, 0)),
            pl.BlockSpec((8, 96), lambda j: (0, 0)),
        ],
        out_shape=[
            jax.ShapeDtypeStruct((E, 96), jnp.float32),
            jax.ShapeDtypeStruct((8, 96), jnp.float32),
        ],
        scratch_shapes=[pltpu.VMEM((8, 96), jnp.float32)],
    )(gs, gn, nbr, wsT, wnT, wfT)


def _passB_body(fc_ref, st_ref, gb_ref, msg_ref):
    st = st_ref[...]
    mean = st[0:1, :] * (1.0 / E)
    var = st[1:2, :] * (1.0 / E) - mean * mean
    inv = lax.rsqrt(var + EPS)
    sc = gb_ref[0:1, :] * inv
    sh = gb_ref[1:2, :] - mean * sc
    r = fc_ref[...] * sc + sh
    f = _sigmoid(r[:, :ATOM])
    c = _softplus(r[:, ATOM:])
    m = f * c
    blk = m.shape[0]
    msg_ref[...] = jnp.concatenate(
        [m, jnp.ones((blk, 1), jnp.float32), jnp.zeros((blk, 7), jnp.float32)],
        axis=1)


def _passB(fc, st, gb):
    grid = E // BE
    return pl.pallas_call(
        _passB_body,
        grid=(grid,),
        in_specs=[
            pl.BlockSpec((BE, 96), lambda j: (j, 0)),
            pl.BlockSpec((8, 96), lambda j: (0, 0)),
            pl.BlockSpec((8, 96), lambda j: (0, 0)),
        ],
        out_specs=pl.BlockSpec((BE, W56), lambda j: (j, 0)),
        out_shape=jax.ShapeDtypeStruct((E, W56), jnp.float32),
    )(fc, st, gb)


def _node_body(s_ref, atom_ref, aux_ref, o_ref, acc_ref, *, nrows, outw):
    p = pl.program_id(0)
    j = pl.program_id(1)
    s = s_ref[...]
    summ = s[:, :ATOM] / jnp.maximum(s[:, ATOM:ATOM + 1], 1.0)

    @pl.when((p == 0) & (j == 0))
    def _():
        acc_ref[...] = jnp.zeros_like(acc_ref)

    @pl.when(p == 0)
    def _():
        acc_ref[0:1, :] += jnp.sum(summ, axis=0, keepdims=True)
        acc_ref[1:2, :] += jnp.sum(summ * summ, axis=0, keepdims=True)

    @pl.when(p == 1)
    def _():
        mean = acc_ref[0:1, :] * (1.0 / nrows)
        var = acc_ref[1:2, :] * (1.0 / nrows) - mean * mean
        inv = lax.rsqrt(var + EPS)
        sc = aux_ref[0:1, :] * inv
        sh = aux_ref[1:2, :] - mean * sc
        y = _softplus(atom_ref[...] + summ * sc + sh)
        if outw == W56:
            blk = y.shape[0]
            o_ref[...] = jnp.concatenate(
                [y, jnp.ones((blk, 1), jnp.float32),
                 jnp.zeros((blk, 7), jnp.float32)], axis=1)
        else:
            o_ref[...] = y


def _node(sums56, atom, aux, outw):
    grid = N // BN_NODE
    return pl.pallas_call(
        functools.partial(_node_body, nrows=N, outw=outw),
        grid=(2, grid),
        in_specs=[
            pl.BlockSpec((BN_NODE, W56), lambda p, j: (j, 0)),
            pl.BlockSpec((BN_NODE, ATOM), lambda p, j: (j, 0)),
            pl.BlockSpec((8, ATOM), lambda p, j: (0, 0)),
        ],
        out_specs=pl.BlockSpec((BN_NODE, outw),
                               lambda p, j: (jnp.where(p == 1, j, 0), 0)),
        out_shape=jax.ShapeDtypeStruct((N, outw), jnp.float32),
        scratch_shapes=[pltpu.VMEM((8, ATOM), jnp.float32)],
    )(sums56, atom, aux)


def _head_body(s_ref, w_ref, aux_ref, o_ref):
    s = s_ref[...]
    crys = s[:, :ATOM] / jnp.maximum(s[:, ATOM:ATOM + 1], 1.0)
    crys = _softplus(crys)
    y = jnp.dot(crys, w_ref[...], preferred_element_type=jnp.float32)
    m = jnp.mean(y, axis=0, keepdims=True)
    v = jnp.mean(y * y, axis=0, keepdims=True) - m * m
    yb = (y - m) * lax.rsqrt(v + EPS) * aux_ref[0:1, :] + aux_ref[1:2, :]
    yb = _softplus(yb)
    out = jnp.sum(yb * aux_ref[2:3, :], axis=1, keepdims=True) + aux_ref[3:4, 0:1]
    o_ref[...] = out


def _head(csums56, gtfWT, aux):
    return pl.pallas_call(
        _head_body,
        grid=(1,),
        in_specs=[
            pl.BlockSpec((C, W56), lambda j: (0, 0)),
            pl.BlockSpec((ATOM, H), lambda j: (0, 0)),
            pl.BlockSpec((8, H), lambda j: (0, 0)),
        ],
        out_specs=pl.BlockSpec((C, 1), lambda j: (0, 0)),
        out_shape=jax.ShapeDtypeStruct((C, 1), jnp.float32),
    )(csums56, gtfWT, aux)


_NCORES = 2
_NSUB = 16
_NW = _NCORES * _NSUB


def _sc_gather(table, idx_s, idx_n):
    per_w = E // _NW
    nblk = (per_w + 127) // 128
    mesh = plsc.VectorSubcoreMesh(core_axis_name="c", subcore_axis_name="s")

    @functools.partial(
        pl.kernel, mesh=mesh,
        compiler_params=pltpu.CompilerParams(use_tc_tiling_on_sc=False, needs_layout_passes=False),
        out_type=[jax.ShapeDtypeStruct((E, ATOM), jnp.float32),
                  jax.ShapeDtypeStruct((E, ATOM), jnp.float32)],
        scratch_types=[
            pltpu.VMEM((128,), jnp.int32),
            pltpu.VMEM((128,), jnp.int32),
            pltpu.VMEM((128, ATOM), jnp.float32),
            pltpu.VMEM((128, ATOM), jnp.float32),
            pltpu.SemaphoreType.DMA,
            pltpu.SemaphoreType.DMA,
        ],
    )
    def k(tab_hbm, is_hbm, in_hbm, os_hbm, on_hbm,
          iv_s, iv_n, rv_s, rv_n, sem1, sem2):
        wid = lax.axis_index("s") * _NCORES + lax.axis_index("c")
        wlo = wid * per_w
        wend = wlo + per_w - 128

        def body(j, _):
            start = wlo + j * 128
            startc = jnp.minimum(start, wend)
            pltpu.sync_copy(is_hbm.at[pl.ds(startc, 128)], iv_s)
            pltpu.sync_copy(in_hbm.at[pl.ds(startc, 128)], iv_n)
            d1 = pltpu.async_copy(tab_hbm.at[iv_s], rv_s, sem1)
            d2 = pltpu.async_copy(tab_hbm.at[iv_n], rv_n, sem2)
            d1.wait()
            d2.wait()
            pltpu.sync_copy(rv_s, os_hbm.at[pl.ds(startc, 128)])
            pltpu.sync_copy(rv_n, on_hbm.at[pl.ds(startc, 128)])
            return 0

        lax.fori_loop(0, nblk, body, 0)

    return k(table, idx_s, idx_n)


def _sc_scatter(vals, idx, rstarts, zeros_z, m_rows, r_pad, nchunks, chunk_rows):
    stripe = chunk_rows // _NSUB
    zb = min(stripe, 200)
    nz = stripe // zb
    wob = 160 if stripe % 160 == 0 else stripe
    nwo = stripe // wob
    mesh = plsc.VectorSubcoreMesh(core_axis_name="c", subcore_axis_name="s")

    @functools.partial(
        pl.kernel, mesh=mesh,
        compiler_params=pltpu.CompilerParams(use_tc_tiling_on_sc=False, needs_layout_passes=False),
        out_type=jax.ShapeDtypeStruct((r_pad, W56), jnp.float32),
        scratch_types=[
            pltpu.VMEM_SHARED((chunk_rows + 8, W56), jnp.float32),
            pltpu.VMEM((16,), jnp.int32),
            pltpu.VMEM((128,), jnp.int32),
            pltpu.VMEM((128,), jnp.int32),
            pltpu.VMEM((128, W56), jnp.float32),
            pltpu.VMEM((zb, W56), jnp.float32),
            pltpu.VMEM((wob, W56), jnp.float32),
        ],
    )
    def k(vals_hbm, idx_hbm, rs_hbm, z_hbm, out_hbm,
          acc, rs_v, iv, lv, mv, zv, wv):
        core = lax.axis_index("c")
        sub = lax.axis_index("s")
        pltpu.sync_copy(rs_hbm, rs_v)
        pltpu.sync_copy(z_hbm.at[pl.ds(0, zb)], zv)
        rs = rs_v[...]
        lanes = lax.iota(jnp.int32, 16)

        for c in range(nchunks):
            base = c * chunk_rows
            s_c = jnp.sum(jnp.where(lanes == c, rs, 0))
            e_c = jnp.sum(jnp.where(lanes == c + 1, rs, 0))

            @pl.when(core == (c % _NCORES))
            def _():
                for z in range(nz):
                    pltpu.sync_copy(zv, acc.at[pl.ds(sub * stripe + z * zb, zb)])

                @pl.when(sub == _NSUB - 1)
                def _():
                    pltpu.sync_copy(zv.at[pl.ds(0, 8)],
                                    acc.at[pl.ds(chunk_rows, 8)])

                plsc.subcore_barrier()

                span = e_c - s_c
                t_lo = s_c + (span * sub) // _NSUB
                t_hi = s_c + (span * (sub + 1)) // _NSUB
                alo = (t_lo // 8) * 8
                nb = (t_hi - alo + 127) // 128

                def body(j, _):
                    start = alo + j * 128
                    startc = jnp.minimum(start, m_rows - 128)
                    pltpu.sync_copy(idx_hbm.at[pl.ds(startc, 128)], iv)
                    pltpu.sync_copy(vals_hbm.at[pl.ds(startc, 128)], mv)
                    lo_eff = jnp.maximum(t_lo, start)
                    for g in range(8):
                        pos = startc + g * 16 + lanes
                        val = iv[pl.ds(g * 16, 16)]
                        ok = (pos >= lo_eff) & (pos < t_hi)
                        loc = jnp.where(ok, val - base, chunk_rows)
                        lv[pl.ds(g * 16, 16)] = loc
                    pltpu.sync_copy(mv, acc.at[lv], add=True)
                    return 0

                lax.fori_loop(0, nb, body, 0)
                plsc.subcore_barrier()

                for z in range(nwo):
                    off = sub * stripe + z * wob
                    pltpu.sync_copy(acc.at[pl.ds(off, wob)], wv)
                    pltpu.sync_copy(wv, out_hbm.at[pl.ds(base + off, wob)])

    return k(vals, idx, rstarts, zeros_z)


NCHUNK_N = 4
CHUNK_N = 25600
RPAD_N = NCHUNK_N * CHUNK_N
NCHUNK_C = 2
CHUNK_C = 128
RPAD_C = NCHUNK_C * CHUNK_C


def kernel(orig_atom_fea, nbr_fea, self_fea_idx, nbr_fea_idx, crystal_atom_idx,
           emb_W, emb_b, filt_W, filt_b, core_W, core_b,
           bnf_g, bnf_b, bnc_g, bnc_b, bno_g, bno_b,
           gtf_W, gtf_b, gtfbn_g, gtfbn_b, out_W, out_b):
    f32 = jnp.float32
    embWT = emb_W.T
    emb_b8 = jnp.zeros((8, ATOM), f32).at[0].set(emb_b)

    wsT, wnT, wfT, gb, naux = [], [], [], [], []
    for i in range(NG):
        Wall = jnp.concatenate([filt_W[i], core_W[i]], axis=0)
        wsT.append(Wall[:, :ATOM].T)
        wnT.append(Wall[:, ATOM:2 * ATOM].T)
        wfT.append(Wall[:, 2 * ATOM:].T)
        g = jnp.concatenate([bnf_g[i], bnc_g[i]])
        b = jnp.concatenate([bnf_b[i], bnc_b[i]])
        gb.append(jnp.zeros((8, 96), f32).at[0].set(g).at[1].set(b))
        naux.append(jnp.zeros((8, ATOM), f32).at[0].set(bno_g[i]).at[1].set(bno_b[i]))

    haux = (jnp.zeros((8, H), f32).at[0].set(gtfbn_g).at[1].set(gtfbn_b)
            .at[2].set(out_W[0]).at[3].set(jnp.full((H,), out_b[0], f32)))

    nb_bounds = jnp.arange(0, NCHUNK_N + 1, dtype=jnp.int32) * CHUNK_N
    rs_n = jnp.full((16,), E, jnp.int32).at[:NCHUNK_N + 1].set(
        jnp.searchsorted(self_fea_idx, nb_bounds).astype(jnp.int32))
    cb_bounds = jnp.arange(0, NCHUNK_C + 1, dtype=jnp.int32) * CHUNK_C
    rs_c = jnp.full((16,), N, jnp.int32).at[:NCHUNK_C + 1].set(
        jnp.searchsorted(crystal_atom_idx, cb_bounds).astype(jnp.int32))
    zeros_z = jnp.zeros((200, W56), f32)

    atom = _embed(orig_atom_fea, embWT, emb_b8)
    for i in range(NG):
        gs, gn = _sc_gather(atom, self_fea_idx, nbr_fea_idx)
        fc, st = _passA(gs, gn, nbr_fea, wsT[i], wnT[i], wfT[i])
        msg = _passB(fc, st, gb[i])
        sums = _sc_scatter(msg, self_fea_idx, rs_n, zeros_z,
                           E, RPAD_N, NCHUNK_N, CHUNK_N)
        atom = _node(sums, atom, naux[i], W56 if i == NG - 1 else ATOM)

    csums = _sc_scatter(atom, crystal_atom_idx, rs_c, zeros_z,
                        N, RPAD_C, NCHUNK_C, CHUNK_C)
    return _head(csums, gtf_W.T, haux)
